# baseline (speedup 1.0000x reference)
import functools

import jax
import jax.numpy as jnp
from jax import lax
from jax.experimental import pallas as pl
from jax.experimental.pallas import tpu as pltpu
from jax.experimental.pallas import tpu_sc as plsc

B = 16384
D = 64
NC = 2
NS = 16
NW = NC * NS
BPW = B // NW
NCHUNK = 4
CW = BPW // NCHUNK
NG = BPW // 16

_mesh = plsc.VectorSubcoreMesh(core_axis_name="c", subcore_axis_name="s")


@functools.partial(
    pl.kernel,
    out_type=jax.ShapeDtypeStruct((B,), jnp.float32),
    mesh=_mesh,
    compiler_params=pltpu.CompilerParams(
        needs_layout_passes=False, use_tc_tiling_on_sc=False),
    scratch_types=[
        pltpu.VMEM((NCHUNK, CW), jnp.int32),
        pltpu.VMEM((NCHUNK, CW), jnp.int32),
        pltpu.VMEM((BPW, D), jnp.float32),
        pltpu.VMEM((BPW, D), jnp.float32),
        pltpu.VMEM((BPW,), jnp.float32),
        pltpu.SemaphoreType.DMA,
    ],
)
def _sc_score(ent_hbm, type_hbm, ent_emb_hbm, type_emb_hbm, out_hbm,
              eidx, tidx, erows, trows, outv, sem):
    wid = lax.axis_index("s") * NC + lax.axis_index("c")
    base = wid * BPW

    for c in range(NCHUNK):
        pltpu.sync_copy(ent_hbm.at[pl.ds(base + c * CW, CW)], eidx.at[c])
        pltpu.sync_copy(type_hbm.at[pl.ds(base + c * CW, CW)], tidx.at[c])

    copies = []
    for c in range(NCHUNK):
        copies.append(pltpu.async_copy(
            ent_emb_hbm.at[eidx.at[c]], erows.at[pl.ds(c * CW, CW)], sem))
        copies.append(pltpu.async_copy(
            type_emb_hbm.at[tidx.at[c]], trows.at[pl.ds(c * CW, CW)], sem))
    for cp in copies:
        cp.wait()

    iota = lax.iota(jnp.int32, 16)

    def group(g, carry):
        r0 = g * 16
        svec = jnp.zeros((16,), jnp.float32)
        for u in range(16):
            r = r0 + u
            p = jnp.zeros((16,), jnp.float32)
            for s in range(D // 16):
                ev = erows[r, pl.ds(s * 16, 16)]
                tv = trows[r, pl.ds(s * 16, 16)]
                p = p + ev * tv
            svec = jnp.where(iota == u, jnp.sum(p), svec)
        outv[pl.ds(r0, 16)] = svec
        return carry

    lax.fori_loop(0, NG, group, 0)

    pltpu.sync_copy(outv, out_hbm.at[pl.ds(base, BPW)])


def kernel(ent, ent_type, batch_type, ent_emb, type_emb):
    del batch_type
    score = _sc_score(ent.astype(jnp.int32), ent_type.astype(jnp.int32),
                      ent_emb, type_emb)
    return score[:, None]


# speedup vs baseline: 1.0223x; 1.0223x over previous
import functools

import jax
import jax.numpy as jnp
from jax import lax
from jax.experimental import pallas as pl
from jax.experimental.pallas import tpu as pltpu
from jax.experimental.pallas import tpu_sc as plsc

B = 16384
D = 64
NT = 1000
NC = 2
NS = 16
NW = NC * NS
BPW = B // NW

_mesh = plsc.VectorSubcoreMesh(core_axis_name="c", subcore_axis_name="s")


@functools.partial(
    pl.kernel,
    out_type=jax.ShapeDtypeStruct((B,), jnp.float32),
    mesh=_mesh,
    compiler_params=pltpu.CompilerParams(
        needs_layout_passes=False, use_tc_tiling_on_sc=True),
    scratch_types=[
        pltpu.VMEM((D, NT), jnp.float32),
        pltpu.VMEM((D, 128), jnp.float32),
        pltpu.VMEM((BPW,), jnp.float32),
        pltpu.VMEM((4, 128), jnp.int32),
        pltpu.VMEM((BPW,), jnp.int32),
        pltpu.VMEM((BPW,), jnp.int32),
        pltpu.SemaphoreType.DMA,
        pltpu.SemaphoreType.DMA,
    ],
)
def _sc_score(ent_hbm, type_hbm, embt_hbm, typet_hbm, out_hbm,
              ttab, tbuf, outv, bidx, eidx_v, tidx_v, sem, sem2):
    wid = lax.axis_index("s") * NC + lax.axis_index("c")
    base = wid * BPW

    pltpu.sync_copy(ent_hbm.at[pl.ds(base, BPW)], eidx_v)
    pltpu.sync_copy(type_hbm.at[pl.ds(base, BPW)], tidx_v)
    pltpu.sync_copy(typet_hbm, ttab)

    iota = lax.iota(jnp.int32, 16)
    for r in range(4):
        for k in range(8):
            bidx[r, pl.ds(16 * k, 16)] = base + r * 128 + 16 * k + iota

    def group(g, carry):
        svec = jnp.zeros((16,), jnp.float32)
        ev16 = eidx_v[pl.ds(g * 16, 16)]
        tv16 = tidx_v[pl.ds(g * 16, 16)]
        for u in range(16):
            e = ev16[u]
            kc = e // 128
            ec = e - kc * 128
            off = pl.multiple_of(kc * 128, 128)
            pltpu.async_copy(embt_hbm.at[:, pl.ds(off, 128)], tbuf, sem).wait()
            tc = tv16[u]
            acc = jnp.zeros((16,), jnp.float32)
            for c in range(D // 16):
                fv = iota + 16 * c
                ev = plsc.load_gather(tbuf, [fv, jnp.full((16,), ec, jnp.int32)])
                tv = plsc.load_gather(ttab, [fv, jnp.full((16,), tc, jnp.int32)])
                acc = acc + ev * tv
            svec = jnp.where(iota == u, jnp.sum(acc), svec)
        outv[pl.ds(g * 16, 16)] = svec
        return carry

    lax.fori_loop(0, BPW // 16, group, 0)

    copies = []
    for r in range(4):
        copies.append(pltpu.async_copy(
            outv.at[pl.ds(r * 128, 128)], out_hbm.at[bidx.at[r]], sem2))
    for cp in copies:
        cp.wait()


def kernel(ent, ent_type, batch_type, ent_emb, type_emb):
    del batch_type
    score = _sc_score(ent.astype(jnp.int32), ent_type.astype(jnp.int32),
                      ent_emb.T, type_emb.T)
    return score[:, None]
